# Initial kernel scaffold; baseline (speedup 1.0000x reference)
import jax
import jax.numpy as jnp
from jax.experimental import pallas as pl


def kernel(uncompressed, mask, codebook):
    raise NotImplementedError("write your pallas kernel here")


# speedup vs baseline: 1.3049x; 1.3049x over previous
import jax
import jax.numpy as jnp
from jax.experimental import pallas as pl
from jax.experimental.pallas import tpu as pltpu
from jax.experimental.pallas import tpu_sc as plsc

B, N, D = 16, 576, 256
K = 8192
T = B * N
M_TILE = 256
N_TILES = T // M_TILE
GATHER_WINDOW = 128
GATHER_STEPS = T // GATHER_WINDOW


def _argmin_body(x_ref, cb_ref, idx_ref, c2_ref):
    i = pl.program_id(0)

    @pl.when(i == 0)
    def _():
        cb0 = cb_ref[...]
        c2_ref[...] = jnp.sum(cb0 * cb0, axis=1).reshape(1, K)

    x = x_ref[...]
    xc = jax.lax.dot_general(
        x, cb_ref[...], (((1,), (1,)), ((), ())),
        preferred_element_type=jnp.float32)
    x2 = jnp.sum(x * x, axis=1, keepdims=True)
    dist = x2 - 2.0 * xc + c2_ref[...]
    mn = jnp.min(dist, axis=1, keepdims=True)
    iota = jax.lax.broadcasted_iota(jnp.int32, dist.shape, 1)
    idx = jnp.min(jnp.where(dist == mn, iota, jnp.int32(K)), axis=1)
    idx_ref[0, 0, :] = idx


def _nearest_idx(x_flat, codebook):
    out = pl.pallas_call(
        _argmin_body,
        grid=(N_TILES,),
        in_specs=[
            pl.BlockSpec((M_TILE, D), lambda i: (i, 0)),
            pl.BlockSpec((K, D), lambda i: (0, 0)),
        ],
        out_specs=pl.BlockSpec((1, 1, M_TILE), lambda i: (i, 0, 0)),
        out_shape=jax.ShapeDtypeStruct((N_TILES, 1, M_TILE), jnp.int32),
        scratch_shapes=[pltpu.VMEM((1, K), jnp.float32)],
    )(x_flat, codebook)
    return out.reshape(T)


def _sc_gather(codebook, idx):
    idx2 = idx.reshape(1, T)
    mesh = plsc.VectorSubcoreMesh(
        core_axis_name="core", subcore_axis_name="subcore")

    @pl.kernel(out_type=jax.ShapeDtypeStruct((T, D), codebook.dtype),
               mesh=mesh)
    def kern(cb_hbm, i_hbm, o_hbm):
        def body(i_vmem, o_vmem):
            pltpu.sync_copy(cb_hbm.at[i_vmem.at[0]], o_vmem)

        pltpu.emit_pipeline(
            body,
            grid=(GATHER_STEPS,),
            in_specs=[pl.BlockSpec((1, GATHER_WINDOW),
                                   index_map=lambda i: (0, i))],
            out_specs=[pl.BlockSpec((GATHER_WINDOW, D),
                                    index_map=lambda i: (i, 0))],
            core_axis_name=("core", "subcore"),
            dimension_semantics=(pltpu.PARALLEL,),
        )(i_hbm, o_hbm)

    return kern(codebook, idx2)


def kernel(uncompressed, mask, codebook):
    x_flat = uncompressed.reshape(T, D)
    idx = _nearest_idx(x_flat, codebook)
    compressed = _sc_gather(codebook, idx).reshape(B, N, D)
    return (compressed, uncompressed, mask, codebook)
